# Initial kernel scaffold; baseline (speedup 1.0000x reference)
import jax
import jax.numpy as jnp
from jax.experimental import pallas as pl


def kernel(x, edge_index, edge_attr, c1_eW, c1_eb, c1_W1, c1_b1, c1_W2, c1_b2, c2_eW, c2_eb, c2_W1, c2_b1, c2_W2, c2_b2, mu_W, mu_b, lv_W, lv_b):
    raise NotImplementedError("write your pallas kernel here")


# speedup vs baseline: 1.8324x; 1.8324x over previous
import functools

import jax
import jax.numpy as jnp
from jax import lax
from jax.experimental import pallas as pl
from jax.experimental.pallas import tpu as pltpu
from jax.experimental.pallas import tpu_sc as plsc

N = 10000
E = 160000
D = 256
HH = 128
ED = 16
L = 64
NCORE = 2
NSUB = 16
EPT = E // NSUB
K = 80
CHUNKS = EPT // K
CPTILES = 10
RPT = N // CPTILES
ZROWS = 40


def _prep_body(ea_ref, w1_ref, b1_ref, w2_ref, b2_ref,
               ee1_ref, ee2_ref):
    ea = ea_ref[...]
    m1 = jnp.dot(ea, w1_ref[...], preferred_element_type=jnp.float32) + b1_ref[...]
    m2 = jnp.dot(ea, w2_ref[...], preferred_element_type=jnp.float32) + b2_ref[...]
    ee1_ref[0] = m1[:, :HH]
    ee1_ref[1] = m1[:, HH:]
    ee2_ref[0] = m2[:, :HH]
    ee2_ref[1] = m2[:, HH:]


def _prep(ea, w1, b1, w2, b2):
    grid = 20
    be = E // grid
    return pl.pallas_call(
        _prep_body,
        grid=(grid,),
        in_specs=[
            pl.BlockSpec((be, ED), lambda i: (i, 0)),
            pl.BlockSpec((ED, D), lambda i: (0, 0)),
            pl.BlockSpec((1, D), lambda i: (0, 0)),
            pl.BlockSpec((ED, D), lambda i: (0, 0)),
            pl.BlockSpec((1, D), lambda i: (0, 0)),
        ],
        out_specs=[
            pl.BlockSpec((NCORE, be, HH), lambda i: (0, i, 0)),
            pl.BlockSpec((NCORE, be, HH), lambda i: (0, i, 0)),
        ],
        out_shape=[
            jax.ShapeDtypeStruct((NCORE, E, HH), jnp.float32),
            jax.ShapeDtypeStruct((NCORE, E, HH), jnp.float32),
        ],
    )(ea, w1, b1, w2, b2)


def _split_x_body(x_ref, xh_ref):
    xh_ref[0] = x_ref[:, :HH]
    xh_ref[1] = x_ref[:, HH:]


def _split_x(x):
    grid = 10
    bn = N // grid
    return pl.pallas_call(
        _split_x_body,
        grid=(grid,),
        in_specs=[pl.BlockSpec((bn, D), lambda i: (i, 0))],
        out_specs=[pl.BlockSpec((NCORE, bn, HH), lambda i: (0, i, 0))],
        out_shape=[jax.ShapeDtypeStruct((NCORE, N, HH), jnp.float32)],
    )(x)[0]


def _sc_body(xh, srcr, dstr, ee, out, src_t, dst_t, rows, eeb, zbuf, acc):
    c = lax.axis_index("c")
    s = lax.axis_index("s")

    zv = jnp.zeros((16,), jnp.float32)

    @pl.when(s < CPTILES)
    def _zero():
        def zrow(i, carry):
            for q in range(HH // 16):
                zbuf[i, pl.ds(q * 16, 16)] = zv
            return carry

        lax.fori_loop(0, ZROWS, zrow, 0)
        for t in range(RPT // ZROWS):
            pltpu.sync_copy(zbuf, acc.at[pl.ds(s * RPT + t * ZROWS, ZROWS)])

    cn = c * N

    plsc.subcore_barrier()

    base_e = s * EPT

    def chunk(j, carry):
        pltpu.sync_copy(srcr.at[s, j], src_t)
        pltpu.sync_copy(dstr.at[s, j], dst_t)
        for q in range(K // 16):
            sl = pl.ds(q * 16, 16)
            src_t[sl] = src_t[sl] + cn
        pltpu.sync_copy(xh.at[src_t], rows)
        pltpu.sync_copy(ee.at[pl.ds(c * E + base_e + j * K, K)], eeb)

        def erow(k, carry2):
            for q in range(HH // 16):
                sl = pl.ds(q * 16, 16)
                rows[k, sl] = jnp.maximum(rows[k, sl] + eeb[k, sl], 0.0)
            return carry2

        lax.fori_loop(0, K, erow, 0)
        pltpu.sync_copy(rows, acc.at[dst_t], add=True)
        return carry

    lax.fori_loop(0, CHUNKS, chunk, 0)

    plsc.subcore_barrier()

    @pl.when(s < CPTILES)
    def _copy_out():
        pltpu.sync_copy(acc.at[pl.ds(s * RPT, RPT)],
                        out.at[pl.ds(cn + s * RPT, RPT)])


@functools.cache
def _sc_layer():
    mesh = plsc.VectorSubcoreMesh(core_axis_name="c", subcore_axis_name="s")
    return pl.kernel(
        _sc_body,
        out_type=jax.ShapeDtypeStruct((NCORE * N, HH), jnp.float32),
        mesh=mesh,
        scratch_types=[
            pltpu.VMEM((K,), jnp.int32),
            pltpu.VMEM((K,), jnp.int32),
            pltpu.VMEM((K, HH), jnp.float32),
            pltpu.VMEM((K, HH), jnp.float32),
            pltpu.VMEM((ZROWS, HH), jnp.float32),
            pltpu.VMEM_SHARED((N, HH), jnp.float32),
        ],
    )


def _mlp_body(xh_ref, agg_ref, w1_ref, b1_ref, w2_ref, b2_ref, yh_ref):
    h = jnp.concatenate(
        [xh_ref[0] + agg_ref[0], xh_ref[1] + agg_ref[1]], axis=1)
    t = jnp.maximum(
        jnp.dot(h, w1_ref[...], preferred_element_type=jnp.float32)
        + b1_ref[...], 0.0)
    y = jnp.dot(t, w2_ref[...], preferred_element_type=jnp.float32) + b2_ref[...]
    y = jnp.where(y >= 0.0, y, 0.1 * y)
    yh_ref[0] = y[:, :HH]
    yh_ref[1] = y[:, HH:]


def _mlp(xh, agg, w1, b1, w2, b2):
    grid = 5
    bn = N // grid
    return pl.pallas_call(
        _mlp_body,
        grid=(grid,),
        in_specs=[
            pl.BlockSpec((NCORE, bn, HH), lambda i: (0, i, 0)),
            pl.BlockSpec((NCORE, bn, HH), lambda i: (0, i, 0)),
            pl.BlockSpec((D, 2 * D), lambda i: (0, 0)),
            pl.BlockSpec((1, 2 * D), lambda i: (0, 0)),
            pl.BlockSpec((2 * D, D), lambda i: (0, 0)),
            pl.BlockSpec((1, D), lambda i: (0, 0)),
        ],
        out_specs=[pl.BlockSpec((NCORE, bn, HH), lambda i: (0, i, 0))],
        out_shape=[jax.ShapeDtypeStruct((NCORE, N, HH), jnp.float32)],
    )(xh, agg, w1, b1, w2, b2)[0]


def _mlp_heads_body(xh_ref, agg_ref, w1_ref, b1_ref, w2_ref, b2_ref,
                    muw_ref, mub_ref, lvw_ref, lvb_ref, mu_ref, lv_ref):
    h = jnp.concatenate(
        [xh_ref[0] + agg_ref[0], xh_ref[1] + agg_ref[1]], axis=1)
    t = jnp.maximum(
        jnp.dot(h, w1_ref[...], preferred_element_type=jnp.float32)
        + b1_ref[...], 0.0)
    y = jnp.dot(t, w2_ref[...], preferred_element_type=jnp.float32) + b2_ref[...]
    y = jnp.where(y >= 0.0, y, 0.1 * y)
    mu_ref[...] = (
        jnp.dot(y, muw_ref[...], preferred_element_type=jnp.float32)
        + mub_ref[...])
    lv_ref[...] = (
        jnp.dot(y, lvw_ref[...], preferred_element_type=jnp.float32)
        + lvb_ref[...])


def _mlp_heads(xh, agg, w1, b1, w2, b2, muw, mub, lvw, lvb):
    grid = 5
    bn = N // grid
    return pl.pallas_call(
        _mlp_heads_body,
        grid=(grid,),
        in_specs=[
            pl.BlockSpec((NCORE, bn, HH), lambda i: (0, i, 0)),
            pl.BlockSpec((NCORE, bn, HH), lambda i: (0, i, 0)),
            pl.BlockSpec((D, 2 * D), lambda i: (0, 0)),
            pl.BlockSpec((1, 2 * D), lambda i: (0, 0)),
            pl.BlockSpec((2 * D, D), lambda i: (0, 0)),
            pl.BlockSpec((1, D), lambda i: (0, 0)),
            pl.BlockSpec((D, L), lambda i: (0, 0)),
            pl.BlockSpec((1, L), lambda i: (0, 0)),
            pl.BlockSpec((D, L), lambda i: (0, 0)),
            pl.BlockSpec((1, L), lambda i: (0, 0)),
        ],
        out_specs=[
            pl.BlockSpec((bn, L), lambda i: (i, 0)),
            pl.BlockSpec((bn, L), lambda i: (i, 0)),
        ],
        out_shape=[
            jax.ShapeDtypeStruct((N, L), jnp.float32),
            jax.ShapeDtypeStruct((N, L), jnp.float32),
        ],
    )(xh, agg, w1, b1, w2, b2, muw, mub, lvw, lvb)


def kernel(x, edge_index, edge_attr,
           c1_eW, c1_eb, c1_W1, c1_b1, c1_W2, c1_b2,
           c2_eW, c2_eb, c2_W1, c2_b1, c2_W2, c2_b2,
           mu_W, mu_b, lv_W, lv_b):
    srcr = edge_index[0].reshape(NSUB, CHUNKS, K)
    dstr = edge_index[1].reshape(NSUB, CHUNKS, K)

    ee1, ee2 = _prep(edge_attr, c1_eW, c1_eb.reshape(1, D),
                     c2_eW, c2_eb.reshape(1, D))
    xh = _split_x(x)

    sc = _sc_layer()
    agg1 = sc(xh.reshape(NCORE * N, HH), srcr, dstr,
              ee1.reshape(NCORE * E, HH))
    yh = _mlp(xh, agg1.reshape(NCORE, N, HH),
              c1_W1, c1_b1.reshape(1, 2 * D), c1_W2, c1_b2.reshape(1, D))

    agg2 = sc(yh.reshape(NCORE * N, HH), srcr, dstr,
              ee2.reshape(NCORE * E, HH))
    mu, lv = _mlp_heads(yh, agg2.reshape(NCORE, N, HH),
                        c2_W1, c2_b1.reshape(1, 2 * D),
                        c2_W2, c2_b2.reshape(1, D),
                        mu_W, mu_b.reshape(1, L), lv_W, lv_b.reshape(1, L))
    return (mu, lv)


# speedup vs baseline: 2.0212x; 1.1030x over previous
import functools

import jax
import jax.numpy as jnp
from jax import lax
from jax.experimental import pallas as pl
from jax.experimental.pallas import tpu as pltpu
from jax.experimental.pallas import tpu_sc as plsc

N = 10000
E = 160000
D = 256
HH = 128
HW = HH // 2
ED = 16
L = 64
NCORE = 2
NSUB = 16
EPT = E // NSUB
K = 80
CHUNKS = EPT // K
CPTILES = 10
RPT = N // CPTILES


def _pack_pair(lo, hi):
    lb = lax.bitcast_convert_type(lo, jnp.int32)
    hb = lax.bitcast_convert_type(hi, jnp.int32)
    lr = lax.shift_right_logical(
        lb + 0x7FFF + (lax.shift_right_logical(lb, 16) & 1), 16)
    hr = lax.shift_right_logical(
        hb + 0x7FFF + (lax.shift_right_logical(hb, 16) & 1), 16)
    return lr | lax.shift_left(hr, 16)


def _pack_halves(m):
    return (_pack_pair(m[:, 0:HW], m[:, HW:HH]),
            _pack_pair(m[:, HH:HH + HW], m[:, HH + HW:]))


def _prep_body(ea_ref, w1_ref, b1_ref, w2_ref, b2_ref,
               ee1_ref, ee2_ref):
    ea = ea_ref[...]
    m1 = jnp.dot(ea, w1_ref[...], preferred_element_type=jnp.float32) + b1_ref[...]
    m2 = jnp.dot(ea, w2_ref[...], preferred_element_type=jnp.float32) + b2_ref[...]
    ee1_ref[0], ee1_ref[1] = _pack_halves(m1)
    ee2_ref[0], ee2_ref[1] = _pack_halves(m2)


def _prep(ea, w1, b1, w2, b2):
    grid = 20
    be = E // grid
    return pl.pallas_call(
        _prep_body,
        grid=(grid,),
        in_specs=[
            pl.BlockSpec((be, ED), lambda i: (i, 0)),
            pl.BlockSpec((ED, D), lambda i: (0, 0)),
            pl.BlockSpec((1, D), lambda i: (0, 0)),
            pl.BlockSpec((ED, D), lambda i: (0, 0)),
            pl.BlockSpec((1, D), lambda i: (0, 0)),
        ],
        out_specs=[
            pl.BlockSpec((NCORE, be, HW), lambda i: (0, i, 0)),
            pl.BlockSpec((NCORE, be, HW), lambda i: (0, i, 0)),
        ],
        out_shape=[
            jax.ShapeDtypeStruct((NCORE, E, HW), jnp.int32),
            jax.ShapeDtypeStruct((NCORE, E, HW), jnp.int32),
        ],
    )(ea, w1, b1, w2, b2)


def _split_x_body(x_ref, xh_ref):
    xh_ref[0], xh_ref[1] = _pack_halves(x_ref[...])


def _split_x(x):
    grid = 10
    bn = N // grid
    return pl.pallas_call(
        _split_x_body,
        grid=(grid,),
        in_specs=[pl.BlockSpec((bn, D), lambda i: (i, 0))],
        out_specs=[pl.BlockSpec((NCORE, bn, HW), lambda i: (0, i, 0))],
        out_shape=[jax.ShapeDtypeStruct((NCORE, N, HW), jnp.int32)],
    )(x)[0]


def _sc_body(xh, srcr, dstr, ee, out,
             sv0, dv0, sv1, dv1, xb0, eb0, xb1, eb1, msg, acc,
             si0, si1, sg0, sg1, se0, se1):
    c = lax.axis_index("c")
    s = lax.axis_index("s")
    cn = c * N

    sv = (sv0, sv1)
    dv = (dv0, dv1)
    xb = (xb0, xb1)
    ebf = (eb0, eb1)
    sig = (si0, si1)
    sgg = (sg0, sg1)
    seg = (se0, se1)

    zv = jnp.zeros((16,), jnp.float32)

    def zrow(i, carry):
        for q in range(HH // 16):
            msg[i, pl.ds(q * 16, 16)] = zv
        return carry

    lax.fori_loop(0, K, zrow, 0)

    @pl.when(s < CPTILES)
    def _zero():
        base = s * RPT
        for t in range(RPT // K):
            pltpu.sync_copy(msg, acc.at[pl.ds(base + t * K, K)])
        rem = RPT - (RPT // K) * K
        if rem:
            pltpu.sync_copy(msg.at[pl.ds(0, rem)],
                            acc.at[pl.ds(base + (RPT // K) * K, rem)])

    base_e = c * E + s * EPT

    def issue_idx(j, b):
        pltpu.async_copy(srcr.at[s, j], sv[b], sig[b])
        pltpu.async_copy(dstr.at[s, j], dv[b], sig[b])

    def wait_idx(b):
        pltpu.make_async_copy(srcr.at[s, 0], sv[b], sig[b]).wait()
        pltpu.make_async_copy(dstr.at[s, 0], dv[b], sig[b]).wait()

    def issue_gather(j, b):
        for q in range(K // 16):
            sl = pl.ds(q * 16, 16)
            sv[b][sl] = sv[b][sl] + cn
        pltpu.async_copy(xh.at[sv[b]], xb[b], sgg[b])
        pltpu.async_copy(ee.at[pl.ds(base_e + j * K, K)], ebf[b], seg[b])

    def wait_ge(b):
        pltpu.make_async_copy(xh.at[sv[b]], xb[b], sgg[b]).wait()
        pltpu.make_async_copy(ee.at[pl.ds(base_e, K)], ebf[b], seg[b]).wait()

    mask = jnp.int32(-65536)

    def compute_scatter(b):
        xbb = xb[b]
        ebb = ebf[b]

        def erow(k, carry):
            for q in range(HW // 16):
                sl = pl.ds(q * 16, 16)
                w = xbb[k, sl]
                u = ebb[k, sl]
                xlo = lax.bitcast_convert_type(w << 16, jnp.float32)
                xhi = lax.bitcast_convert_type(w & mask, jnp.float32)
                ulo = lax.bitcast_convert_type(u << 16, jnp.float32)
                uhi = lax.bitcast_convert_type(u & mask, jnp.float32)
                msg[k, sl] = jnp.maximum(xlo + ulo, 0.0)
                msg[k, pl.ds(HW + q * 16, 16)] = jnp.maximum(xhi + uhi, 0.0)
            return carry

        lax.fori_loop(0, K, erow, 0)
        pltpu.sync_copy(msg, acc.at[dv[b]], add=True)

    pltpu.sync_copy(srcr.at[s, 0], sv0)
    pltpu.sync_copy(dstr.at[s, 0], dv0)
    issue_gather(0, 0)
    issue_idx(1, 1)
    plsc.subcore_barrier()

    def pair(i, carry):
        j0 = 2 * i
        wait_idx(1)
        issue_gather(j0 + 1, 1)
        wait_ge(0)
        compute_scatter(0)
        issue_idx(j0 + 2, 0)
        wait_idx(0)
        issue_gather(j0 + 2, 0)
        wait_ge(1)
        compute_scatter(1)

        @pl.when(i < (CHUNKS - 1) // 2 - 1)
        def _():
            issue_idx(j0 + 3, 1)

        return carry

    lax.fori_loop(0, (CHUNKS - 1) // 2, pair, 0)
    wait_ge(0)
    compute_scatter(0)

    plsc.subcore_barrier()

    @pl.when(s < CPTILES)
    def _copy_out():
        pltpu.sync_copy(acc.at[pl.ds(s * RPT, RPT)],
                        out.at[pl.ds(cn + s * RPT, RPT)])


@functools.cache
def _sc_layer():
    mesh = plsc.VectorSubcoreMesh(core_axis_name="c", subcore_axis_name="s")
    return pl.kernel(
        _sc_body,
        out_type=jax.ShapeDtypeStruct((NCORE * N, HH), jnp.float32),
        mesh=mesh,
        compiler_params=pltpu.CompilerParams(use_tc_tiling_on_sc=False),
        scratch_types=[
            pltpu.VMEM((K,), jnp.int32),
            pltpu.VMEM((K,), jnp.int32),
            pltpu.VMEM((K,), jnp.int32),
            pltpu.VMEM((K,), jnp.int32),
            pltpu.VMEM((K, HW), jnp.int32),
            pltpu.VMEM((K, HW), jnp.int32),
            pltpu.VMEM((K, HW), jnp.int32),
            pltpu.VMEM((K, HW), jnp.int32),
            pltpu.VMEM((K, HH), jnp.float32),
            pltpu.VMEM_SHARED((N, HH), jnp.float32),
            pltpu.SemaphoreType.DMA,
            pltpu.SemaphoreType.DMA,
            pltpu.SemaphoreType.DMA,
            pltpu.SemaphoreType.DMA,
            pltpu.SemaphoreType.DMA,
            pltpu.SemaphoreType.DMA,
        ],
    )


def _mlp_body(x_ref, agg_ref, w1_ref, b1_ref, w2_ref, b2_ref,
              y_ref, ybf_ref):
    h = jnp.concatenate(
        [x_ref[:, :HH] + agg_ref[0], x_ref[:, HH:] + agg_ref[1]], axis=1)
    t = jnp.maximum(
        jnp.dot(h, w1_ref[...], preferred_element_type=jnp.float32)
        + b1_ref[...], 0.0)
    y = jnp.dot(t, w2_ref[...], preferred_element_type=jnp.float32) + b2_ref[...]
    y = jnp.where(y >= 0.0, y, 0.1 * y)
    y_ref[...] = y
    ybf_ref[0], ybf_ref[1] = _pack_halves(y)


def _mlp(x, agg, w1, b1, w2, b2):
    grid = 5
    bn = N // grid
    return pl.pallas_call(
        _mlp_body,
        grid=(grid,),
        in_specs=[
            pl.BlockSpec((bn, D), lambda i: (i, 0)),
            pl.BlockSpec((NCORE, bn, HH), lambda i: (0, i, 0)),
            pl.BlockSpec((D, 2 * D), lambda i: (0, 0)),
            pl.BlockSpec((1, 2 * D), lambda i: (0, 0)),
            pl.BlockSpec((2 * D, D), lambda i: (0, 0)),
            pl.BlockSpec((1, D), lambda i: (0, 0)),
        ],
        out_specs=[
            pl.BlockSpec((bn, D), lambda i: (i, 0)),
            pl.BlockSpec((NCORE, bn, HW), lambda i: (0, i, 0)),
        ],
        out_shape=[
            jax.ShapeDtypeStruct((N, D), jnp.float32),
            jax.ShapeDtypeStruct((NCORE, N, HW), jnp.int32),
        ],
    )(x, agg, w1, b1, w2, b2)


def _mlp_heads_body(y_ref, agg_ref, w1_ref, b1_ref, w2_ref, b2_ref,
                    muw_ref, mub_ref, lvw_ref, lvb_ref, mu_ref, lv_ref):
    h = jnp.concatenate(
        [y_ref[:, :HH] + agg_ref[0], y_ref[:, HH:] + agg_ref[1]], axis=1)
    t = jnp.maximum(
        jnp.dot(h, w1_ref[...], preferred_element_type=jnp.float32)
        + b1_ref[...], 0.0)
    y = jnp.dot(t, w2_ref[...], preferred_element_type=jnp.float32) + b2_ref[...]
    y = jnp.where(y >= 0.0, y, 0.1 * y)
    mu_ref[...] = (
        jnp.dot(y, muw_ref[...], preferred_element_type=jnp.float32)
        + mub_ref[...])
    lv_ref[...] = (
        jnp.dot(y, lvw_ref[...], preferred_element_type=jnp.float32)
        + lvb_ref[...])


def _mlp_heads(y, agg, w1, b1, w2, b2, muw, mub, lvw, lvb):
    grid = 5
    bn = N // grid
    return pl.pallas_call(
        _mlp_heads_body,
        grid=(grid,),
        in_specs=[
            pl.BlockSpec((bn, D), lambda i: (i, 0)),
            pl.BlockSpec((NCORE, bn, HH), lambda i: (0, i, 0)),
            pl.BlockSpec((D, 2 * D), lambda i: (0, 0)),
            pl.BlockSpec((1, 2 * D), lambda i: (0, 0)),
            pl.BlockSpec((2 * D, D), lambda i: (0, 0)),
            pl.BlockSpec((1, D), lambda i: (0, 0)),
            pl.BlockSpec((D, L), lambda i: (0, 0)),
            pl.BlockSpec((1, L), lambda i: (0, 0)),
            pl.BlockSpec((D, L), lambda i: (0, 0)),
            pl.BlockSpec((1, L), lambda i: (0, 0)),
        ],
        out_specs=[
            pl.BlockSpec((bn, L), lambda i: (i, 0)),
            pl.BlockSpec((bn, L), lambda i: (i, 0)),
        ],
        out_shape=[
            jax.ShapeDtypeStruct((N, L), jnp.float32),
            jax.ShapeDtypeStruct((N, L), jnp.float32),
        ],
    )(y, agg, w1, b1, w2, b2, muw, mub, lvw, lvb)


def kernel(x, edge_index, edge_attr,
           c1_eW, c1_eb, c1_W1, c1_b1, c1_W2, c1_b2,
           c2_eW, c2_eb, c2_W1, c2_b1, c2_W2, c2_b2,
           mu_W, mu_b, lv_W, lv_b):
    srcr = edge_index[0].reshape(NSUB, CHUNKS, K)
    dstr = edge_index[1].reshape(NSUB, CHUNKS, K)

    ee1, ee2 = _prep(edge_attr, c1_eW, c1_eb.reshape(1, D),
                     c2_eW, c2_eb.reshape(1, D))
    xh = _split_x(x)

    sc = _sc_layer()
    agg1 = sc(xh.reshape(NCORE * N, HW), srcr, dstr,
              ee1.reshape(NCORE * E, HW))
    y, ybf = _mlp(x, agg1.reshape(NCORE, N, HH),
                  c1_W1, c1_b1.reshape(1, 2 * D),
                  c1_W2, c1_b2.reshape(1, D))

    agg2 = sc(ybf.reshape(NCORE * N, HW), srcr, dstr,
              ee2.reshape(NCORE * E, HW))
    mu, lv = _mlp_heads(y, agg2.reshape(NCORE, N, HH),
                        c2_W1, c2_b1.reshape(1, 2 * D),
                        c2_W2, c2_b2.reshape(1, D),
                        mu_W, mu_b.reshape(1, L), lv_W, lv_b.reshape(1, L))
    return (mu, lv)
